# baseline (speedup 1.0000x reference)
import jax
import jax.numpy as jnp
from jax import lax
from jax.experimental import pallas as pl
from jax.experimental.pallas import tpu as pltpu
from jax.experimental.pallas import tpu_sc as plsc

_N = 10000
_E = 320000
_D = 128
_NC, _NS, _L = 2, 16, 16
_NW = _NC * _NS
_EPW = _E // _NW
_R_BLK = 320000


def _prep_body(et_ref, ei_ref, h_ref, w1_ref, b1_ref, w2_ref, b2_ref,
               r_ref, s_ref, d_ref, pq_ref):
    w2e = w2_ref[:, _D:]
    r_ref[...] = lax.dot_general(w2e, et_ref[...], (((1,), (0,)), ((), ())),
                                 preferred_element_type=jnp.float32)
    s_ref[...] = ei_ref[0:1, :]
    d_ref[...] = ei_ref[1:2, :]

    @pl.when(pl.program_id(0) == 0)
    def _tables():
        w2h = w2_ref[:, :_D]
        v = lax.dot_general(w2h, w1_ref[...], (((1,), (0,)), ((), ())),
                            preferred_element_type=jnp.float32)
        va = v[:, :_D]
        vb = v[:, _D:]
        h = h_ref[...]
        p = lax.dot_general(va, h, (((1,), (1,)), ((), ())),
                            preferred_element_type=jnp.float32)
        q = lax.dot_general(vb, h, (((1,), (1,)), ((), ())),
                            preferred_element_type=jnp.float32)
        c = lax.dot_general(w2h, b1_ref[...], (((1,), (1,)), ((), ())),
                            preferred_element_type=jnp.float32)
        pq_ref[0:1, :] = p + c + b2_ref[...]
        pq_ref[1:2, :] = q


def _sc_body(pq_hbm, s_hbm, d_hbm, r_hbm, out_hbm,
             p_v, q_v, s_v, d_v, r_v, o_v, sem):
    cid = lax.axis_index("c")
    sid = lax.axis_index("s")
    wid = sid * _NC + cid
    base = wid * _EPW
    cps = [
        pltpu.async_copy(pq_hbm.at[0], p_v, sem),
        pltpu.async_copy(pq_hbm.at[1], q_v, sem),
        pltpu.async_copy(s_hbm.at[0].at[pl.ds(base, _EPW)], s_v, sem),
        pltpu.async_copy(d_hbm.at[0].at[pl.ds(base, _EPW)], d_v, sem),
        pltpu.async_copy(r_hbm.at[0].at[pl.ds(base, _EPW)], r_v, sem),
    ]
    for cp in cps:
        cp.wait()

    @plsc.parallel_loop(0, _EPW, step=_L, unroll=25)
    def _loop(off):
        sidx = s_v[pl.ds(off, _L)]
        didx = d_v[pl.ds(off, _L)]
        pv = plsc.load_gather(p_v, [sidx])
        qv = plsc.load_gather(q_v, [didx])
        rv = r_v[pl.ds(off, _L)]
        o_v[pl.ds(off, _L)] = pv + qv + rv

    pltpu.sync_copy(o_v, out_hbm.at[0].at[pl.ds(base, _EPW)])


@jax.jit
def kernel(h, edge_index, e, W1_w, W1_b, W2_w, W2_b):
    ei = edge_index.astype(jnp.int32)
    b1 = W1_b.reshape(1, _D)
    b2 = W2_b.reshape(1, 1)
    et = e.T

    r, s, d, pq = pl.pallas_call(
        _prep_body,
        grid=(_E // _R_BLK,),
        compiler_params=pltpu.CompilerParams(vmem_limit_bytes=100 * 1024 * 1024),
        in_specs=[
            pl.BlockSpec((16, _R_BLK), lambda i: (0, i)),
            pl.BlockSpec((2, _R_BLK), lambda i: (0, i)),
            pl.BlockSpec((_N, _D), lambda i: (0, 0)),
            pl.BlockSpec((_D, 2 * _D), lambda i: (0, 0)),
            pl.BlockSpec((1, _D), lambda i: (0, 0)),
            pl.BlockSpec((1, 144), lambda i: (0, 0)),
            pl.BlockSpec((1, 1), lambda i: (0, 0)),
        ],
        out_specs=[
            pl.BlockSpec((1, _R_BLK), lambda i: (0, i)),
            pl.BlockSpec((1, _R_BLK), lambda i: (0, i)),
            pl.BlockSpec((1, _R_BLK), lambda i: (0, i)),
            pl.BlockSpec((2, _N), lambda i: (0, 0)),
        ],
        out_shape=[
            jax.ShapeDtypeStruct((1, _E), jnp.float32),
            jax.ShapeDtypeStruct((1, _E), jnp.int32),
            jax.ShapeDtypeStruct((1, _E), jnp.int32),
            jax.ShapeDtypeStruct((2, _N), jnp.float32),
        ],
    )(et, ei, h, W1_w, b1, W2_w, b2)

    mesh = plsc.VectorSubcoreMesh(core_axis_name="c", subcore_axis_name="s")
    score = pl.kernel(
        _sc_body,
        out_type=jax.ShapeDtypeStruct((1, _E), jnp.float32),
        mesh=mesh,
        compiler_params=pltpu.CompilerParams(needs_layout_passes=False),
        scratch_types=[
            pltpu.VMEM((_N,), jnp.float32),
            pltpu.VMEM((_N,), jnp.float32),
            pltpu.VMEM((_EPW,), jnp.int32),
            pltpu.VMEM((_EPW,), jnp.int32),
            pltpu.VMEM((_EPW,), jnp.float32),
            pltpu.VMEM((_EPW,), jnp.float32),
            pltpu.SemaphoreType.DMA,
        ],
    )(pq, s, d, r)

    return score.reshape(_E, 1)


# speedup vs baseline: 1.0760x; 1.0760x over previous
import jax
import jax.numpy as jnp
from jax import lax
from jax.experimental import pallas as pl
from jax.experimental.pallas import tpu as pltpu
from jax.experimental.pallas import tpu_sc as plsc

_N = 10000
_E = 320000
_D = 128
_NC, _NS, _L = 2, 16, 16
_NW = _NC * _NS
_EPW = _E // _NW
_R_BLK = 160000


def _prep_body(et_ref, ei_ref, h_ref, w1_ref, b1_ref, w2_ref, b2_ref,
               r_ref, s_ref, d_ref, pq_ref):
    w2e = w2_ref[:, _D:]
    r_ref[...] = lax.dot_general(w2e, et_ref[...], (((1,), (0,)), ((), ())),
                                 preferred_element_type=jnp.float32)
    s_ref[...] = ei_ref[0:1, :]
    d_ref[...] = ei_ref[1:2, :]

    @pl.when(pl.program_id(0) == 0)
    def _tables():
        w2h = w2_ref[:, :_D]
        v = lax.dot_general(w2h, w1_ref[...], (((1,), (0,)), ((), ())),
                            preferred_element_type=jnp.float32)
        va = v[:, :_D]
        vb = v[:, _D:]
        h = h_ref[...]
        p = lax.dot_general(va, h, (((1,), (1,)), ((), ())),
                            preferred_element_type=jnp.float32)
        q = lax.dot_general(vb, h, (((1,), (1,)), ((), ())),
                            preferred_element_type=jnp.float32)
        c = lax.dot_general(w2h, b1_ref[...], (((1,), (1,)), ((), ())),
                            preferred_element_type=jnp.float32)
        pq_ref[0:1, :] = p + c + b2_ref[...]
        pq_ref[1:2, :] = q


def _sc_body(pq_hbm, s_hbm, d_hbm, r_hbm, out_hbm,
             p_v, q_v, s_v, d_v, r_v, o_v, sem):
    cid = lax.axis_index("c")
    sid = lax.axis_index("s")
    wid = sid * _NC + cid
    base = wid * _EPW
    cps = [
        pltpu.async_copy(pq_hbm.at[0], p_v, sem),
        pltpu.async_copy(pq_hbm.at[1], q_v, sem),
        pltpu.async_copy(s_hbm.at[0].at[pl.ds(base, _EPW)], s_v, sem),
        pltpu.async_copy(d_hbm.at[0].at[pl.ds(base, _EPW)], d_v, sem),
        pltpu.async_copy(r_hbm.at[0].at[pl.ds(base, _EPW)], r_v, sem),
    ]
    for cp in cps:
        cp.wait()

    @plsc.parallel_loop(0, _EPW, step=_L, unroll=25)
    def _loop(off):
        sidx = s_v[pl.ds(off, _L)]
        didx = d_v[pl.ds(off, _L)]
        pv = plsc.load_gather(p_v, [sidx])
        qv = plsc.load_gather(q_v, [didx])
        rv = r_v[pl.ds(off, _L)]
        o_v[pl.ds(off, _L)] = pv + qv + rv

    pltpu.sync_copy(o_v, out_hbm.at[0].at[pl.ds(base, _EPW)])


@jax.jit
def kernel(h, edge_index, e, W1_w, W1_b, W2_w, W2_b):
    ei = edge_index.astype(jnp.int32)
    b1 = W1_b.reshape(1, _D)
    b2 = W2_b.reshape(1, 1)
    et = e.T

    r, s, d, pq = pl.pallas_call(
        _prep_body,
        grid=(_E // _R_BLK,),
        in_specs=[
            pl.BlockSpec((16, _R_BLK), lambda i: (0, i)),
            pl.BlockSpec((2, _R_BLK), lambda i: (0, i)),
            pl.BlockSpec((_N, _D), lambda i: (0, 0)),
            pl.BlockSpec((_D, 2 * _D), lambda i: (0, 0)),
            pl.BlockSpec((1, _D), lambda i: (0, 0)),
            pl.BlockSpec((1, 144), lambda i: (0, 0)),
            pl.BlockSpec((1, 1), lambda i: (0, 0)),
        ],
        out_specs=[
            pl.BlockSpec((1, _R_BLK), lambda i: (0, i)),
            pl.BlockSpec((1, _R_BLK), lambda i: (0, i)),
            pl.BlockSpec((1, _R_BLK), lambda i: (0, i)),
            pl.BlockSpec((2, _N), lambda i: (0, 0)),
        ],
        out_shape=[
            jax.ShapeDtypeStruct((1, _E), jnp.float32),
            jax.ShapeDtypeStruct((1, _E), jnp.int32),
            jax.ShapeDtypeStruct((1, _E), jnp.int32),
            jax.ShapeDtypeStruct((2, _N), jnp.float32),
        ],
    )(et, ei, h, W1_w, b1, W2_w, b2)

    mesh = plsc.VectorSubcoreMesh(core_axis_name="c", subcore_axis_name="s")
    score = pl.kernel(
        _sc_body,
        out_type=jax.ShapeDtypeStruct((1, _E), jnp.float32),
        mesh=mesh,
        compiler_params=pltpu.CompilerParams(
            needs_layout_passes=False, skip_device_barrier=True),
        scratch_types=[
            pltpu.VMEM((_N,), jnp.float32),
            pltpu.VMEM((_N,), jnp.float32),
            pltpu.VMEM((_EPW,), jnp.int32),
            pltpu.VMEM((_EPW,), jnp.int32),
            pltpu.VMEM((_EPW,), jnp.float32),
            pltpu.VMEM((_EPW,), jnp.float32),
            pltpu.SemaphoreType.DMA,
        ],
    )(pq, s, d, r)

    return score.reshape(_E, 1)
